# baseline (speedup 1.0000x reference)
import functools

import jax
import jax.numpy as jnp
from jax import lax
from jax.experimental import pallas as pl
from jax.experimental.pallas import tpu as pltpu
from jax.experimental.pallas import tpu_sc as plsc

C, H, W = 336, 144, 256
N_CAM, N_LVL, BY, BX = 4, 6, 240, 120
CELLS = BY * BX
PERCAM = N_LVL * CELLS
LANES = 16
NW = 32
N_ROUND = (C + NW - 1) // NW

_mesh = plsc.VectorSubcoreMesh(
    core_axis_name="c", subcore_axis_name="s", num_cores=2, num_subcores=16)


QCELLS = CELLS // 4
HCELLS = QCELLS // 2
PW_WORDS = N_CAM * PERCAM
N_TASK = C * 4 // NW
N_BLK = N_CAM * N_LVL * 2


@functools.partial(
    pl.kernel,
    out_type=jax.ShapeDtypeStruct((N_LVL * C * CELLS,), jnp.float32),
    mesh=_mesh,
    compiler_params=pltpu.CompilerParams(needs_layout_passes=False),
    scratch_types=[
        pltpu.VMEM((H * 128,), jnp.int32),
        pltpu.VMEM((H * 128,), jnp.int32),
        pltpu.VMEM((N_LVL * QCELLS,), jnp.float32),
        pltpu.VMEM((HCELLS,), jnp.int32),
        pltpu.VMEM((HCELLS,), jnp.int32),
        pltpu.VMEM_SHARED((PW_WORDS,), jnp.int32),
        pltpu.SemaphoreType.DMA,
        pltpu.SemaphoreType.DMA,
        pltpu.SemaphoreType.DMA,
        pltpu.SemaphoreType.DMA,
        pltpu.SemaphoreType.DMA,
    ],
)
def _sc_project(feat_hbm, u_hbm, v_hbm, va_hbm, de_hbm, out_hbm,
                plane0, plane1, acc, pw0, pw1,
                pw_spm, s_pl0, s_pl1, s_pw0, s_pw1, s_out):
    cid = lax.axis_index("c")
    sid = lax.axis_index("s")
    wid = sid * 2 + cid
    qq = wid % 4
    grp = wid // 4

    planes = (plane0, plane1)
    psems = (s_pl0, s_pl1)
    pws = (pw0, pw1)
    wsems = (s_pw0, s_pw1)

    seg = PW_WORDS // 16

    def stage_body(j, _):
        off = sid * seg + j * HCELLS
        pltpu.sync_copy(u_hbm.at[pl.ds(off, HCELLS)], pw0)
        pltpu.sync_copy(v_hbm.at[pl.ds(off, HCELLS)], pw1)
        pltpu.sync_copy(va_hbm.at[pl.ds(off, HCELLS)], acc.at[pl.ds(0, HCELLS)])
        pltpu.sync_copy(de_hbm.at[pl.ds(off, HCELLS)],
                        acc.at[pl.ds(HCELLS, HCELLS)])

        @plsc.parallel_loop(0, HCELLS, LANES, unroll=4)
        def pack_body(i):
            s = pl.ds(i, LANES)
            u = pw0[s]
            v = pw1[s]
            word = lax.shift_left(v, 7) | lax.shift_right_logical(u, 1)
            sel = lax.shift_left(u & 1, 16)
            w = acc[s] * acc[pl.ds(HCELLS + i, LANES)]
            wb = lax.shift_right_logical(
                plsc.bitcast(w, jnp.int32) + 0x8000, 16)
            pw0[s] = lax.shift_left(word, 17) | sel | wb

        pltpu.sync_copy(pw0, pw_spm.at[pl.ds(off, HCELLS)])
        return 0

    lax.fori_loop(0, seg // HCELLS, stage_body, 0)
    plsc.subcore_barrier()

    def pw_src(b):
        k, r = divmod(b % N_BLK, N_LVL * 2)
        l, h = divmod(r, 2)
        off = (k * N_LVL + l) * CELLS + qq * QCELLS + h * HCELLS
        return pw_spm.at[pl.ds(off, HCELLS)]

    pltpu.async_copy(feat_hbm.at[grp], plane0, s_pl0)
    pltpu.async_copy(pw_src(0), pw0, s_pw0)

    def task(t, _):
        ch = t * 8 + grp

        @pl.when(t > 0)
        def _():
            for l in range(N_LVL):
                pltpu.make_async_copy(
                    acc.at[pl.ds(l * QCELLS, QCELLS)],
                    out_hbm.at[pl.ds((l * C + ch) * CELLS + qq * QCELLS, QCELLS)],
                    s_out).wait()

        for b in range(N_BLK):
            k, r = divmod(b, N_LVL * 2)
            l, h = divmod(r, 2)
            buf = b % 2
            if b % (N_LVL * 2) == 0:
                pltpu.make_async_copy(feat_hbm.at[k * C + ch],
                                      planes[k % 2], psems[k % 2]).wait()
                if k < N_CAM - 1:
                    nxt = (k + 1) * C + ch
                else:
                    nxt = ch + 8
                pltpu.async_copy(feat_hbm.at[nxt], planes[(k + 1) % 2],
                                 psems[(k + 1) % 2])
            pltpu.make_async_copy(pw_src(b), pws[buf], wsems[buf]).wait()
            pltpu.async_copy(pw_src(b + 1), pws[1 - buf], wsems[1 - buf])

            plbuf = planes[k % 2]
            pwbuf = pws[buf]
            base = l * QCELLS + h * HCELLS

            @plsc.parallel_loop(0, HCELLS, LANES, unroll=8)
            def i_body(off):
                s = pl.ds(off, LANES)
                x = pwbuf[s]
                p2 = lax.shift_right_logical(x, 17)
                sh = 16 - lax.shift_right_logical(x & 0x10000, 12)
                w = plsc.bitcast(lax.shift_left(x, 16), jnp.float32)
                g = plsc.load_gather(plbuf, [p2])
                gf = plsc.bitcast(lax.shift_left(g, sh), jnp.float32)
                d = pl.ds(base + off, LANES)
                if k == 0:
                    acc[d] = gf * w
                else:
                    acc[d] = acc[d] + gf * w

        for l in range(N_LVL):
            pltpu.async_copy(
                acc.at[pl.ds(l * QCELLS, QCELLS)],
                out_hbm.at[pl.ds((l * C + ch) * CELLS + qq * QCELLS, QCELLS)],
                s_out)
        return 0

    lax.fori_loop(0, N_TASK, task, 0)

    ch_last = (N_TASK - 1) * 8 + grp
    pltpu.make_async_copy(pw_src(0), pw0, s_pw0).wait()
    pltpu.make_async_copy(feat_hbm.at[ch_last + 8], plane0, s_pl0).wait()
    for l in range(N_LVL):
        pltpu.make_async_copy(
            acc.at[pl.ds(l * QCELLS, QCELLS)],
            out_hbm.at[pl.ds((l * C + ch_last) * CELLS + qq * QCELLS, QCELLS)],
            s_out).wait()


@jax.jit
def kernel(input, projection_u, projection_v, projection_valid,
           projection_density):
    fb = input.astype(jnp.bfloat16).reshape(N_CAM * C, H, W // 2, 2)
    feat = lax.bitcast_convert_type(fb, jnp.uint32).view(jnp.int32).reshape(
        N_CAM * C, H * 128)
    out = _sc_project(feat, projection_u.reshape(PW_WORDS),
                      projection_v.reshape(PW_WORDS),
                      projection_valid.reshape(PW_WORDS),
                      projection_density.reshape(PW_WORDS))
    return out.reshape(1, N_LVL * C, BY, BX)


# speedup vs baseline: 1.3019x; 1.3019x over previous
import functools

import jax
import jax.numpy as jnp
from jax import lax
from jax.experimental import pallas as pl
from jax.experimental.pallas import tpu as pltpu
from jax.experimental.pallas import tpu_sc as plsc

C, H, W = 336, 144, 256
N_CAM, N_LVL, BY, BX = 4, 6, 240, 120
CELLS = BY * BX
PERCAM = N_LVL * CELLS
LANES = 16
NW = 32
N_ROUND = (C + NW - 1) // NW

_mesh = plsc.VectorSubcoreMesh(
    core_axis_name="c", subcore_axis_name="s", num_cores=2, num_subcores=16)


def _pack_body(u_ref, v_ref, va_ref, de_ref, o_ref):
    u = u_ref[...]
    v = v_ref[...]
    word = (v * 128 + (u & 127)).astype(jnp.uint32)
    sel = (u >> 7).astype(jnp.uint32)
    w = va_ref[...] * de_ref[...]
    wb = lax.bitcast_convert_type(w, jnp.uint32)
    wb = (wb + jnp.uint32(0x8000)) & jnp.uint32(0xFFFF0000)
    o_ref[...] = lax.bitcast_convert_type(
        (word << 17) | (sel << 16) | (wb >> 16), jnp.int32)


_PACK_R = PERCAM // 128

_pack_tc = pl.pallas_call(
    _pack_body,
    out_shape=jax.ShapeDtypeStruct((N_CAM, _PACK_R, 128), jnp.int32),
    grid=(N_CAM,),
    in_specs=[pl.BlockSpec((1, _PACK_R, 128), lambda i: (i, 0, 0))] * 4,
    out_specs=pl.BlockSpec((1, _PACK_R, 128), lambda i: (i, 0, 0)),
)


def _featpack_body(x_ref, o_ref):
    xb = lax.bitcast_convert_type(x_ref[...], jnp.uint32)
    rne = (xb + jnp.uint32(0x7FFF) + ((xb >> 16) & jnp.uint32(1))) >> 16
    lo = rne[:, :, :128]
    hi = rne[:, :, 128:]
    o_ref[...] = lax.bitcast_convert_type(lo | (hi << 16), jnp.int32)


_FEAT_B = 8

_featpack_tc = pl.pallas_call(
    _featpack_body,
    out_shape=jax.ShapeDtypeStruct((N_CAM * C, H, 128), jnp.int32),
    grid=(N_CAM * C // _FEAT_B,),
    in_specs=[pl.BlockSpec((_FEAT_B, H, W), lambda i: (i, 0, 0))],
    out_specs=pl.BlockSpec((_FEAT_B, H, 128), lambda i: (i, 0, 0)),
)


QCELLS = CELLS // 4
HCELLS = QCELLS // 2
PW_WORDS = N_CAM * PERCAM
N_TASK = C * 4 // NW
N_BLK = N_CAM * N_LVL * 2


@functools.partial(
    pl.kernel,
    out_type=jax.ShapeDtypeStruct((N_LVL * C * CELLS,), jnp.float32),
    mesh=_mesh,
    compiler_params=pltpu.CompilerParams(needs_layout_passes=False),
    scratch_types=[
        pltpu.VMEM((H * 128,), jnp.int32),
        pltpu.VMEM((H * 128,), jnp.int32),
        pltpu.VMEM((N_LVL * QCELLS,), jnp.float32),
        pltpu.VMEM((HCELLS,), jnp.int32),
        pltpu.VMEM((HCELLS,), jnp.int32),
        pltpu.VMEM_SHARED((PW_WORDS,), jnp.int32),
        pltpu.SemaphoreType.DMA,
        pltpu.SemaphoreType.DMA,
        pltpu.SemaphoreType.DMA,
        pltpu.SemaphoreType.DMA,
        pltpu.SemaphoreType.DMA,
    ],
)
def _sc_project(feat_hbm, pw_hbm, out_hbm, plane0, plane1, acc, pw0, pw1,
                pw_spm, s_pl0, s_pl1, s_pw0, s_pw1, s_out):
    cid = lax.axis_index("c")
    sid = lax.axis_index("s")
    wid = sid * 2 + cid
    qq = wid % 4
    grp = wid // 4

    planes = (plane0, plane1)
    psems = (s_pl0, s_pl1)
    pws = (pw0, pw1)
    wsems = (s_pw0, s_pw1)

    seg = PW_WORDS // 16

    def stage_body(j, _):
        off = sid * seg + j * HCELLS
        pltpu.sync_copy(pw_hbm.at[pl.ds(off, HCELLS)], pw0)
        pltpu.sync_copy(pw0, pw_spm.at[pl.ds(off, HCELLS)])
        return 0

    lax.fori_loop(0, seg // HCELLS, stage_body, 0)
    plsc.subcore_barrier()

    def pw_src(b):
        k, r = divmod(b % N_BLK, N_LVL * 2)
        l, h = divmod(r, 2)
        off = (k * N_LVL + l) * CELLS + qq * QCELLS + h * HCELLS
        return pw_spm.at[pl.ds(off, HCELLS)]

    pltpu.async_copy(feat_hbm.at[grp], plane0, s_pl0)
    pltpu.async_copy(pw_src(0), pw0, s_pw0)

    def task(t, _):
        ch = t * 8 + grp

        @pl.when(t > 0)
        def _():
            for l in range(N_LVL):
                pltpu.make_async_copy(
                    acc.at[pl.ds(l * QCELLS, QCELLS)],
                    out_hbm.at[pl.ds((l * C + ch) * CELLS + qq * QCELLS, QCELLS)],
                    s_out).wait()

        for b in range(N_BLK):
            k, r = divmod(b, N_LVL * 2)
            l, h = divmod(r, 2)
            buf = b % 2
            if b % (N_LVL * 2) == 0:
                pltpu.make_async_copy(feat_hbm.at[k * C + ch],
                                      planes[k % 2], psems[k % 2]).wait()
                if k < N_CAM - 1:
                    nxt = (k + 1) * C + ch
                else:
                    nxt = ch + 8
                pltpu.async_copy(feat_hbm.at[nxt], planes[(k + 1) % 2],
                                 psems[(k + 1) % 2])
            pltpu.make_async_copy(pw_src(b), pws[buf], wsems[buf]).wait()
            pltpu.async_copy(pw_src(b + 1), pws[1 - buf], wsems[1 - buf])

            plbuf = planes[k % 2]
            pwbuf = pws[buf]
            base = l * QCELLS + h * HCELLS

            @plsc.parallel_loop(0, HCELLS, LANES, unroll=8)
            def i_body(off):
                s = pl.ds(off, LANES)
                x = pwbuf[s]
                p2 = lax.shift_right_logical(x, 17)
                sh = 16 - lax.shift_right_logical(x & 0x10000, 12)
                w = plsc.bitcast(lax.shift_left(x, 16), jnp.float32)
                g = plsc.load_gather(plbuf, [p2])
                gf = plsc.bitcast(lax.shift_left(g, sh), jnp.float32)
                d = pl.ds(base + off, LANES)
                if k == 0:
                    acc[d] = gf * w
                else:
                    acc[d] = acc[d] + gf * w

        for l in range(N_LVL):
            pltpu.async_copy(
                acc.at[pl.ds(l * QCELLS, QCELLS)],
                out_hbm.at[pl.ds((l * C + ch) * CELLS + qq * QCELLS, QCELLS)],
                s_out)
        return 0

    lax.fori_loop(0, N_TASK, task, 0)

    ch_last = (N_TASK - 1) * 8 + grp
    pltpu.make_async_copy(pw_src(0), pw0, s_pw0).wait()
    pltpu.make_async_copy(feat_hbm.at[ch_last + 8], plane0, s_pl0).wait()
    for l in range(N_LVL):
        pltpu.make_async_copy(
            acc.at[pl.ds(l * QCELLS, QCELLS)],
            out_hbm.at[pl.ds((l * C + ch_last) * CELLS + qq * QCELLS, QCELLS)],
            s_out).wait()


@jax.jit
def kernel(input, projection_u, projection_v, projection_valid,
           projection_density):
    feat = _featpack_tc(input.reshape(N_CAM * C, H, W)).reshape(
        N_CAM * C, H * 128)
    u2 = projection_u.reshape(N_CAM, _PACK_R, 128)
    v2 = projection_v.reshape(N_CAM, _PACK_R, 128)
    va2 = projection_valid.reshape(N_CAM, _PACK_R, 128)
    de2 = projection_density.reshape(N_CAM, _PACK_R, 128)
    pw = _pack_tc(u2, v2, va2, de2).reshape(PW_WORDS)
    out = _sc_project(feat, pw)
    return out.reshape(1, N_LVL * C, BY, BX)
